# baseline (speedup 1.0000x reference)
import numpy as np
import jax
import jax.numpy as jnp
from jax import lax
from jax.experimental import pallas as pl
from jax.experimental.pallas import tpu as pltpu

_B = 512
_M = 512
_N = 2048
_L = 16
_RHO = 0.5
_K = 64
_NBLK = 4
_BN = _N // _NBLK
_INF_BITS = 0x7F800000


def _ancestor_matrix() -> np.ndarray:
    A = np.zeros((_N, _N // 2), np.float32)
    for i in range(_N):
        j = i
        while j != 0:
            j = (j - 1) // 2
            A[i, j] = 1.0
    return A


def _body(y_ref, w1_ref, w2_ref, disc_ref, th_ref, anc_ref, out_ref, u_scr):
    k = pl.program_id(0)
    j = pl.program_id(1)

    u_blk = lax.dot_general(
        y_ref[...], w1_ref[0], (((1,), (1,)), ((), ())),
        preferred_element_type=jnp.float32)

    @pl.when(k == 0)
    def _():
        u_scr[:, pl.ds(j * _BN, _BN)] = u_blk

    @pl.when(k > 0)
    def _():
        u2 = lax.dot_general(
            out_ref[...], w2_ref[0], (((1,), (1,)), ((), ())),
            preferred_element_type=jnp.float32)
        u_scr[:, pl.ds(j * _BN, _BN)] = u_blk + u2

    @pl.when(j == _NBLK - 1)
    def _():
        u = u_scr[...]
        s = jnp.abs(u) * disc_ref[...]
        sb = lax.bitcast_convert_type(s, jnp.int32)

        def bis(_, carry):
            lo, hi = carry
            d = hi - lo
            q = lax.shift_right_logical(d, 2)
            m1 = lo + q
            m2 = lo + lax.shift_right_logical(d, 1)
            m3 = m2 + q

            def count_ge(mv):
                return jnp.sum((sb >= mv).astype(jnp.float32), axis=1,
                               keepdims=True)

            big1 = count_ge(m1) >= float(_K)
            big2 = count_ge(m2) >= float(_K)
            big3 = count_ge(m3) >= float(_K)
            nlo = jnp.where(big3, m3, jnp.where(big2, m2,
                            jnp.where(big1, m1, lo)))
            nhi = jnp.where(big3, hi, jnp.where(big2, m3,
                            jnp.where(big1, m2, m1)))
            return nlo, nhi

        lo0 = jnp.zeros((_B, 1), jnp.int32)
        hi0 = jnp.full((_B, 1), _INF_BITS, jnp.int32)
        vk, _hi = lax.fori_loop(0, 16, bis, (lo0, hi0))

        selb = sb >= vk
        sel = selb.astype(jnp.bfloat16)
        hits = lax.dot_general(
            sel, anc_ref[...], (((1,), (0,)), ((), ())),
            preferred_element_type=jnp.float32)
        anc_hit = jnp.concatenate(
            [hits, jnp.zeros((_B, _N // 2), jnp.float32)], axis=1) > 0.0
        mask = (selb | anc_hit).astype(jnp.float32)

        th = jnp.abs(th_ref[0])[0:1, 0:1]
        out_ref[...] = jnp.sign(u) * jnp.maximum(jnp.abs(u) - th, 0.0) * mask


def kernel(y, W1, W2, thresholds, parent, depth):
    del parent
    disc = (_RHO ** depth.astype(jnp.float32)).reshape(1, _N)
    th3 = jnp.broadcast_to(
        thresholds.astype(jnp.float32).reshape(_L, 1, 1), (_L, 1, 128))
    anc = jnp.asarray(_ancestor_matrix(), dtype=jnp.bfloat16)

    return pl.pallas_call(
        _body,
        grid=(_L, _NBLK),
        in_specs=[
            pl.BlockSpec((_B, _M), lambda k, j: (0, 0)),
            pl.BlockSpec((1, _BN, _M), lambda k, j: (k, j, 0)),
            pl.BlockSpec((1, _BN, _N), lambda k, j: (k, j, 0)),
            pl.BlockSpec((1, _N), lambda k, j: (0, 0)),
            pl.BlockSpec((1, 1, 128), lambda k, j: (k, 0, 0)),
            pl.BlockSpec((_N, _N // 2), lambda k, j: (0, 0)),
        ],
        out_specs=pl.BlockSpec((_B, _N), lambda k, j: (0, 0)),
        out_shape=jax.ShapeDtypeStruct((_B, _N), jnp.float32),
        scratch_shapes=[pltpu.VMEM((_B, _N), jnp.float32)],
        compiler_params=pltpu.CompilerParams(
            dimension_semantics=("arbitrary", "arbitrary")),
    )(y, W1, W2, disc, th3, anc)


# speedup vs baseline: 1.1565x; 1.1565x over previous
import numpy as np
import jax
import jax.numpy as jnp
from jax import lax
from jax.experimental import pallas as pl
from jax.experimental.pallas import tpu as pltpu

_B = 512
_M = 512
_N = 2048
_L = 16
_RHO = 0.5
_K = 64
_NBLK = 4
_BN = _N // _NBLK
_INF_BITS = 0x7F800000


def _ancestor_matrix() -> np.ndarray:
    A = np.zeros((_N, _N // 2), np.float32)
    for i in range(_N):
        j = i
        while j != 0:
            j = (j - 1) // 2
            A[i, j] = 1.0
    return A


def _body(y_ref, w1_ref, w2_ref, disc_ref, th_ref, anc_ref, out_ref, u_scr):
    k = pl.program_id(0)
    j = pl.program_id(1)

    u_blk = lax.dot_general(
        y_ref[...], w1_ref[0], (((1,), (1,)), ((), ())),
        preferred_element_type=jnp.float32)

    @pl.when(k == 0)
    def _():
        u_scr[:, pl.ds(j * _BN, _BN)] = u_blk

    @pl.when(k > 0)
    def _():
        u2 = lax.dot_general(
            out_ref[...], w2_ref[0], (((1,), (1,)), ((), ())),
            preferred_element_type=jnp.float32)
        u_scr[:, pl.ds(j * _BN, _BN)] = u_blk + u2

    @pl.when(j == _NBLK - 1)
    def _():
        u = u_scr[...]
        s = jnp.abs(u) * disc_ref[...]
        sb = lax.bitcast_convert_type(s, jnp.int32)

        def bis(_, carry):
            lo, hi = carry
            mid = lo + lax.shift_right_logical(hi - lo, 1)
            cnt = jnp.sum((sb >= mid).astype(jnp.float32), axis=1,
                          keepdims=True)
            big = cnt >= float(_K)
            return jnp.where(big, mid, lo), jnp.where(big, hi, mid)

        lo0 = jnp.zeros((_B, 1), jnp.int32)
        hi0 = jnp.full((_B, 1), _INF_BITS, jnp.int32)
        vk, _hi = lax.fori_loop(0, 31, bis, (lo0, hi0))

        selb = sb >= vk
        sel = selb.astype(jnp.bfloat16)
        hits = lax.dot_general(
            sel, anc_ref[...], (((1,), (0,)), ((), ())),
            preferred_element_type=jnp.float32)
        anc_hit = jnp.concatenate(
            [hits, jnp.zeros((_B, _N // 2), jnp.float32)], axis=1) > 0.0
        mask = (selb | anc_hit).astype(jnp.float32)

        th = jnp.abs(th_ref[0])[0:1, 0:1]
        out_ref[...] = jnp.sign(u) * jnp.maximum(jnp.abs(u) - th, 0.0) * mask


def kernel(y, W1, W2, thresholds, parent, depth):
    del parent
    disc = (_RHO ** depth.astype(jnp.float32)).reshape(1, _N)
    th3 = jnp.broadcast_to(
        thresholds.astype(jnp.float32).reshape(_L, 1, 1), (_L, 1, 128))
    anc = jnp.asarray(_ancestor_matrix(), dtype=jnp.bfloat16)

    return pl.pallas_call(
        _body,
        grid=(_L, _NBLK),
        in_specs=[
            pl.BlockSpec((_B, _M), lambda k, j: (0, 0)),
            pl.BlockSpec((1, _BN, _M), lambda k, j: (k, j, 0)),
            pl.BlockSpec((1, _BN, _N), lambda k, j: (k, j, 0)),
            pl.BlockSpec((1, _N), lambda k, j: (0, 0)),
            pl.BlockSpec((1, 1, 128), lambda k, j: (k, 0, 0)),
            pl.BlockSpec((_N, _N // 2), lambda k, j: (0, 0)),
        ],
        out_specs=pl.BlockSpec((_B, _N), lambda k, j: (0, 0)),
        out_shape=jax.ShapeDtypeStruct((_B, _N), jnp.float32),
        scratch_shapes=[pltpu.VMEM((_B, _N), jnp.float32)],
        compiler_params=pltpu.CompilerParams(
            dimension_semantics=("arbitrary", "arbitrary")),
    )(y, W1, W2, disc, th3, anc)


# speedup vs baseline: 1.2120x; 1.0480x over previous
import numpy as np
import jax
import jax.numpy as jnp
from jax import lax
from jax.experimental import pallas as pl
from jax.experimental.pallas import tpu as pltpu

_B = 512
_M = 512
_N = 2048
_L = 16
_RHO = 0.5
_K = 64
_NBLK = 2
_BN = _N // _NBLK
_INF_BITS = 0x7F800000


def _ancestor_matrix() -> np.ndarray:
    A = np.zeros((_N, _N // 2), np.float32)
    for i in range(_N):
        j = i
        while j != 0:
            j = (j - 1) // 2
            A[i, j] = 1.0
    return A


def _body(y_ref, w1_ref, w2_ref, disc_ref, th_ref, anc_ref, out_ref, u_scr):
    k = pl.program_id(0)
    j = pl.program_id(1)

    u_blk = lax.dot_general(
        y_ref[...], w1_ref[0], (((1,), (1,)), ((), ())),
        preferred_element_type=jnp.float32)

    @pl.when(k == 0)
    def _():
        u_scr[:, pl.ds(j * _BN, _BN)] = u_blk

    @pl.when(k > 0)
    def _():
        u2 = lax.dot_general(
            out_ref[...], w2_ref[0], (((1,), (1,)), ((), ())),
            preferred_element_type=jnp.float32)
        u_scr[:, pl.ds(j * _BN, _BN)] = u_blk + u2

    @pl.when(j == _NBLK - 1)
    def _():
        u = u_scr[...]
        s = jnp.abs(u) * disc_ref[...]
        sb = lax.bitcast_convert_type(s, jnp.int32)

        def bis(_, carry):
            lo, hi = carry
            mid = lo + lax.shift_right_logical(hi - lo, 1)
            cnt = jnp.sum((sb >= mid).astype(jnp.float32), axis=1,
                          keepdims=True)
            big = cnt >= float(_K)
            return jnp.where(big, mid, lo), jnp.where(big, hi, mid)

        lo0 = jnp.zeros((_B, 1), jnp.int32)
        hi0 = jnp.full((_B, 1), _INF_BITS, jnp.int32)
        vk, _hi = lax.fori_loop(0, 31, bis, (lo0, hi0))

        selb = sb >= vk
        sel = selb.astype(jnp.bfloat16)
        hits = lax.dot_general(
            sel, anc_ref[...], (((1,), (0,)), ((), ())),
            preferred_element_type=jnp.float32)
        anc_hit = jnp.concatenate(
            [hits, jnp.zeros((_B, _N // 2), jnp.float32)], axis=1) > 0.0
        mask = (selb | anc_hit).astype(jnp.float32)

        th = jnp.abs(th_ref[0])[0:1, 0:1]
        out_ref[...] = jnp.sign(u) * jnp.maximum(jnp.abs(u) - th, 0.0) * mask


def kernel(y, W1, W2, thresholds, parent, depth):
    del parent
    disc = (_RHO ** depth.astype(jnp.float32)).reshape(1, _N)
    th3 = jnp.broadcast_to(
        thresholds.astype(jnp.float32).reshape(_L, 1, 1), (_L, 1, 128))
    anc = jnp.asarray(_ancestor_matrix(), dtype=jnp.bfloat16)

    return pl.pallas_call(
        _body,
        grid=(_L, _NBLK),
        in_specs=[
            pl.BlockSpec((_B, _M), lambda k, j: (0, 0)),
            pl.BlockSpec((1, _BN, _M), lambda k, j: (k, j, 0)),
            pl.BlockSpec((1, _BN, _N), lambda k, j: (k, j, 0)),
            pl.BlockSpec((1, _N), lambda k, j: (0, 0)),
            pl.BlockSpec((1, 1, 128), lambda k, j: (k, 0, 0)),
            pl.BlockSpec((_N, _N // 2), lambda k, j: (0, 0)),
        ],
        out_specs=pl.BlockSpec((_B, _N), lambda k, j: (0, 0)),
        out_shape=jax.ShapeDtypeStruct((_B, _N), jnp.float32),
        scratch_shapes=[pltpu.VMEM((_B, _N), jnp.float32)],
        compiler_params=pltpu.CompilerParams(
            dimension_semantics=("arbitrary", "arbitrary")),
    )(y, W1, W2, disc, th3, anc)
